# Initial kernel scaffold; baseline (speedup 1.0000x reference)
import jax
import jax.numpy as jnp
from jax.experimental import pallas as pl


def kernel(x, edge_index, edge_features, W1, b1, W2, b2, W3, b3, gamma, beta):
    raise NotImplementedError("write your pallas kernel here")


# speedup vs baseline: 2.6698x; 2.6698x over previous
import functools

import jax
import jax.numpy as jnp
from jax import lax
from jax.experimental import pallas as pl
from jax.experimental.pallas import tpu as pltpu
from jax.experimental.pallas import tpu_sc as plsc

N_NODES = 10000
N_EDGES = 320000
DIM = 128
EDGE_DIM = 16

NC = 2
NS = 16
NW = NC * NS
EPW = N_EDGES // NW
CHUNK = 80
NCHUNKS = EPW // CHUNK
RPT = 624
TAIL = N_NODES - NS * RPT


def _xw1_body(x_ref, w_ref, b_ref, o_ref):
    o_ref[...] = (
        jnp.dot(x_ref[...], w_ref[...], preferred_element_type=jnp.float32)
        + b_ref[...]
    )


def _efw_body(e_ref, w_ref, o_ref):
    o_ref[...] = jnp.dot(e_ref[...], w_ref[...],
                         preferred_element_type=jnp.float32)


def _final_body(x_ref, a2_ref, c_ref, w2_ref, b2_ref, w3x_ref, w3a_ref,
                b3_ref, g_ref, be_ref, o_ref):
    agg_h = a2_ref[0] + a2_ref[1]
    cnt = jnp.sum(c_ref[...], axis=1, keepdims=True)
    agg = (jnp.dot(agg_h, w2_ref[...], preferred_element_type=jnp.float32)
           + cnt * b2_ref[...]) / (cnt + 1e-8)
    u = jnp.dot(x_ref[...], w3x_ref[...], preferred_element_type=jnp.float32)
    u = u + jnp.dot(agg, w3a_ref[...], preferred_element_type=jnp.float32)
    u = u + b3_ref[...]
    y = jnp.maximum(u, 0.0) + x_ref[...]
    m = jnp.mean(y, axis=1, keepdims=True)
    v = jnp.mean(jnp.square(y - m), axis=1, keepdims=True)
    y = (y - m) * lax.rsqrt(v + 1e-5)
    o_ref[...] = y * g_ref[...] + be_ref[...]


def _edge_body(xw1, efw, src, dst, zacc,
               agg_out, cnt_out,
               src_v, dst_v, rows_v, ef_v, cnt_v, acc_s, sem):
    c = lax.axis_index("c")
    s = lax.axis_index("s")
    wid = s * NC + c

    pltpu.sync_copy(zacc.at[pl.ds(s * RPT, RPT)], acc_s.at[pl.ds(s * RPT, RPT)])

    @pl.when(s == NS - 1)
    def _zero_tail():
        pltpu.sync_copy(zacc.at[pl.ds(NS * RPT, TAIL)],
                        acc_s.at[pl.ds(NS * RPT, TAIL)])

    zero16 = jnp.zeros((16,), jnp.float32)

    def zero_body(i, _):
        cnt_v[pl.ds(i * 16, 16)] = zero16
        return ()

    lax.fori_loop(0, N_NODES // 16, zero_body, (), unroll=8)
    plsc.subcore_barrier()

    ebase = wid * EPW
    one16 = jnp.full((16,), 1.0, jnp.float32)

    def chunk_body(t, _):
        base = ebase + t * CHUNK
        pltpu.sync_copy(src.at[pl.ds(base, CHUNK)], src_v)
        pltpu.sync_copy(dst.at[pl.ds(base, CHUNK)], dst_v)
        pltpu.sync_copy(efw.at[pl.ds(base, CHUNK)], ef_v)
        pltpu.async_copy(xw1.at[src_v], rows_v, sem).wait()

        def relu_body(i, _):
            for j in range(DIM // 16):
                sl = pl.ds(j * 16, 16)
                v = rows_v[i, sl] + ef_v[i, sl]
                rows_v[i, sl] = jnp.maximum(v, 0.0)
            return ()

        lax.fori_loop(0, CHUNK, relu_body, ())

        pltpu.sync_copy(rows_v, acc_s.at[dst_v], add=True)

        def cnt_body(k, _):
            idx = dst_v[pl.ds(k * 16, 16)]
            plsc.addupdate_scatter(cnt_v, [idx], one16)
            return ()

        lax.fori_loop(0, CHUNK // 16, cnt_body, ())
        return ()

    lax.fori_loop(0, NCHUNKS, chunk_body, ())
    plsc.subcore_barrier()

    pltpu.sync_copy(acc_s.at[pl.ds(s * RPT, RPT)],
                    agg_out.at[c, pl.ds(s * RPT, RPT)])

    @pl.when(s == NS - 1)
    def _write_tail():
        pltpu.sync_copy(acc_s.at[pl.ds(NS * RPT, TAIL)],
                        agg_out.at[c, pl.ds(NS * RPT, TAIL)])

    pltpu.sync_copy(cnt_v, cnt_out.at[pl.ds(wid * N_NODES, N_NODES)])


def kernel(x, edge_index, edge_features, W1, b1, W2, b2, W3, b3, gamma, beta):
    x = x.astype(jnp.float32)
    src = edge_index[0].astype(jnp.int32)
    dst = edge_index[1].astype(jnp.int32)

    w1xT = W1[:, :DIM].T
    w1eT = W1[:, DIM:].T
    wbig = jax.scipy.linalg.block_diag(*([w1eT] * 8))
    w2T = W2.T
    w3xT = W3[:, :DIM].T
    w3aT = W3[:, DIM:].T
    b1r = b1.reshape(1, DIM)
    b2r = b2.reshape(1, DIM)
    b3r = b3.reshape(1, DIM)
    gr = gamma.reshape(1, DIM)
    br = beta.reshape(1, DIM)

    xw1 = pl.pallas_call(
        _xw1_body,
        grid=(5,),
        in_specs=[
            pl.BlockSpec((2000, DIM), lambda i: (i, 0)),
            pl.BlockSpec((DIM, DIM), lambda i: (0, 0)),
            pl.BlockSpec((1, DIM), lambda i: (0, 0)),
        ],
        out_specs=pl.BlockSpec((2000, DIM), lambda i: (i, 0)),
        out_shape=jax.ShapeDtypeStruct((N_NODES, DIM), jnp.float32),
    )(x, w1xT, b1r)

    ef2 = edge_features.reshape(N_EDGES // 8, 128)
    efw2 = pl.pallas_call(
        _efw_body,
        grid=(40,),
        in_specs=[
            pl.BlockSpec((1000, 128), lambda i: (i, 0)),
            pl.BlockSpec((128, 8 * DIM), lambda i: (0, 0)),
        ],
        out_specs=pl.BlockSpec((1000, 8 * DIM), lambda i: (i, 0)),
        out_shape=jax.ShapeDtypeStruct((N_EDGES // 8, 8 * DIM), jnp.float32),
    )(ef2, wbig)
    efw = efw2.reshape(N_EDGES, DIM)

    zacc = jnp.zeros((N_NODES, DIM), jnp.float32)

    agg2, cnt_flat = pl.kernel(
        _edge_body,
        out_type=(
            jax.ShapeDtypeStruct((NC, N_NODES, DIM), jnp.float32),
            jax.ShapeDtypeStruct((NW * N_NODES,), jnp.float32),
        ),
        mesh=plsc.VectorSubcoreMesh(core_axis_name="c", subcore_axis_name="s"),
        compiler_params=pltpu.CompilerParams(needs_layout_passes=False),
        scratch_types=[
            pltpu.VMEM((CHUNK,), jnp.int32),
            pltpu.VMEM((CHUNK,), jnp.int32),
            pltpu.VMEM((CHUNK, DIM), jnp.float32),
            pltpu.VMEM((CHUNK, DIM), jnp.float32),
            pltpu.VMEM((N_NODES,), jnp.float32),
            pltpu.VMEM_SHARED((N_NODES, DIM), jnp.float32),
            pltpu.SemaphoreType.DMA,
        ],
    )(xw1, efw, src, dst, zacc)
    cnt32 = cnt_flat.reshape(NW, N_NODES)

    y = pl.pallas_call(
        _final_body,
        grid=(5,),
        in_specs=[
            pl.BlockSpec((2000, DIM), lambda i: (i, 0)),
            pl.BlockSpec((NC, 2000, DIM), lambda i: (0, i, 0)),
            pl.BlockSpec((2000, NW), lambda i: (i, 0)),
            pl.BlockSpec((DIM, DIM), lambda i: (0, 0)),
            pl.BlockSpec((1, DIM), lambda i: (0, 0)),
            pl.BlockSpec((DIM, DIM), lambda i: (0, 0)),
            pl.BlockSpec((DIM, DIM), lambda i: (0, 0)),
            pl.BlockSpec((1, DIM), lambda i: (0, 0)),
            pl.BlockSpec((1, DIM), lambda i: (0, 0)),
            pl.BlockSpec((1, DIM), lambda i: (0, 0)),
        ],
        out_specs=pl.BlockSpec((2000, DIM), lambda i: (i, 0)),
        out_shape=jax.ShapeDtypeStruct((N_NODES, DIM), jnp.float32),
    )(x, agg2, cnt32.T, w2T, b2r, w3xT, w3aT, b3r, gr, br)

    return y
